# baseline (speedup 1.0000x reference)
import functools
import math

import jax
import jax.numpy as jnp
from jax import lax
from jax.experimental import pallas as pl
from jax.experimental.pallas import tpu as pltpu
from jax.experimental.pallas import tpu_sc as plsc

D_MODEL = 64
SCALE = math.sqrt(D_MODEL)

G = 128
K = 4
C = G * K


@functools.lru_cache(maxsize=None)
def _build(B):
    info = plsc.get_sparse_core_info()
    NW = info.num_cores * info.num_subcores
    assert B % (NW * C) == 0
    b_per_w = B // NW
    n_chunks = b_per_w // C
    g_rows_per_w = b_per_w // G

    mesh = plsc.VectorSubcoreMesh(core_axis_name="c", subcore_axis_name="s")

    @functools.partial(
        pl.kernel,
        mesh=mesh,
        compiler_params=pltpu.CompilerParams(use_tc_tiling_on_sc=False),
        out_type=jax.ShapeDtypeStruct((B, D_MODEL), jnp.float32),
        scratch_types=[
            pltpu.VMEM((K, G), jnp.int32),
            pltpu.VMEM((C, D_MODEL), jnp.float32),
            pltpu.SemaphoreType.DMA,
        ],
    )
    def emb_kernel(idx_hbm, table_hbm, out_hbm, idx_v, rows_v, sem):
        cid = lax.axis_index("c")
        sid = lax.axis_index("s")
        wid = sid * info.num_cores + cid
        row_base = wid * b_per_w
        gidx_base = wid * g_rows_per_w

        def chunk_body(ci, carry):
            row0 = row_base + ci * C
            pltpu.sync_copy(idx_hbm.at[pl.ds(gidx_base + ci * K, K)], idx_v)
            copies = [
                pltpu.async_copy(
                    table_hbm.at[idx_v.at[j]],
                    rows_v.at[pl.ds(j * G, G)],
                    sem,
                )
                for j in range(K)
            ]
            for cp in copies:
                cp.wait()

            def scale_row(r, carry2):
                for d in range(D_MODEL // 16):
                    sl = pl.ds(d * 16, 16)
                    rows_v[r, sl] = rows_v[r, sl] * SCALE
                return carry2

            lax.fori_loop(0, C, scale_row, 0, unroll=2)

            pltpu.sync_copy(rows_v, out_hbm.at[pl.ds(row0, C)])
            return carry

        lax.fori_loop(0, n_chunks, chunk_body, 0)

    return emb_kernel


def kernel(x, pretrained_vector):
    B = x.shape[0] * x.shape[1]
    idx2d = x.reshape(B // G, G).astype(jnp.int32)
    out = _build(B)(idx2d, pretrained_vector)
    return out.reshape(x.shape[0], x.shape[1], D_MODEL)


# speedup vs baseline: 1.0895x; 1.0895x over previous
import functools
import math

import jax
import jax.numpy as jnp
from jax import lax
from jax.experimental import pallas as pl
from jax.experimental.pallas import tpu as pltpu
from jax.experimental.pallas import tpu_sc as plsc

D_MODEL = 64
SCALE = math.sqrt(D_MODEL)

G = 128
K = 2
C = G * K
NBUF = 4


@functools.lru_cache(maxsize=None)
def _build(B):
    info = plsc.get_sparse_core_info()
    NW = info.num_cores * info.num_subcores
    assert B % (NW * C * NBUF) == 0
    b_per_w = B // NW
    n_chunks = b_per_w // C
    n_groups = n_chunks // NBUF
    g_rows_per_w = b_per_w // G

    mesh = plsc.VectorSubcoreMesh(core_axis_name="c", subcore_axis_name="s")

    @functools.partial(
        pl.kernel,
        mesh=mesh,
        compiler_params=pltpu.CompilerParams(use_tc_tiling_on_sc=False),
        out_type=jax.ShapeDtypeStruct((B, D_MODEL), jnp.float32),
        scratch_types=[
            pltpu.VMEM((g_rows_per_w, G), jnp.int32),
            pltpu.VMEM((NBUF, C, D_MODEL), jnp.float32),
            pltpu.SemaphoreType.DMA,
            pltpu.SemaphoreType.DMA,
            pltpu.SemaphoreType.DMA,
            pltpu.SemaphoreType.DMA,
            pltpu.SemaphoreType.DMA,
            pltpu.SemaphoreType.DMA,
            pltpu.SemaphoreType.DMA,
            pltpu.SemaphoreType.DMA,
        ],
    )
    def emb_kernel(idx_hbm, table_hbm, out_hbm, idx_v, rows_v, *sems):
        gsem = sems[:NBUF]
        osem = sems[NBUF:]
        cid = lax.axis_index("c")
        sid = lax.axis_index("s")
        wid = sid * info.num_cores + cid
        row_base = wid * b_per_w

        pltpu.sync_copy(idx_hbm.at[pl.ds(wid * g_rows_per_w, g_rows_per_w)],
                        idx_v)

        def fire_gather(ci, s):
            for j in range(K):
                pltpu.async_copy(
                    table_hbm.at[idx_v.at[ci * K + j]],
                    rows_v.at[s, pl.ds(j * G, G)],
                    gsem[s],
                )

        def wait_gather(s):
            pltpu.make_async_copy(
                table_hbm.at[pl.ds(0, C)], rows_v.at[s], gsem[s]).wait()

        def wait_out(ci, s):
            pltpu.make_async_copy(
                rows_v.at[s],
                out_hbm.at[pl.ds(row_base + ci * C, C)],
                osem[s],
            ).wait()

        for b in range(NBUF - 1):
            fire_gather(b, b)

        def group_body(g, carry):
            for b in range(NBUF):
                ci = g * NBUF + b
                wait_gather(b)

                def scale_row(r, carry2):
                    for d in range(D_MODEL // 16):
                        sl = pl.ds(d * 16, 16)
                        rows_v[b, r, sl] = rows_v[b, r, sl] * SCALE
                    return carry2

                lax.fori_loop(0, C, scale_row, 0, unroll=8)

                pltpu.async_copy(
                    rows_v.at[b],
                    out_hbm.at[pl.ds(row_base + ci * C, C)],
                    osem[b],
                )

                s2 = (b + NBUF - 1) % NBUF
                @pl.when(ci >= 1)
                def _():
                    wait_out(ci - 1, s2)

                @pl.when(ci + NBUF - 1 < n_chunks)
                def _():
                    fire_gather(ci + NBUF - 1, s2)
            return carry

        lax.fori_loop(0, n_groups, group_body, 0)
        wait_out(n_chunks - 1, (n_chunks - 1) % NBUF)

    return emb_kernel


def kernel(x, pretrained_vector):
    B = x.shape[0] * x.shape[1]
    idx2d = x.reshape(B // G, G).astype(jnp.int32)
    out = _build(B)(idx2d, pretrained_vector)
    return out.reshape(x.shape[0], x.shape[1], D_MODEL)
